# baseline (speedup 1.0000x reference)
import jax
import jax.numpy as jnp
from jax.experimental import pallas as pl

_B, _C, _T = 16, 128, 1500
_BINS = 2048
_CDIM = 32
_NQ = 8
_TT = 1500
_NT = _T // _TT
_NGRP = _BINS // 128

_MM = jax.lax.Precision.DEFAULT


def _mm(a, b):
    return jax.lax.dot_general(a, b, (((1,), (0,)), ((), ())),
                               precision=_MM,
                               preferred_element_type=jnp.float32)


def _gather_rows(eT, idx):
    rows = eT.shape[0]
    r = jnp.broadcast_to(idx % 128, (rows, idx.shape[1]))
    g = idx // 128
    acc = None
    for gi in range(_NGRP):
        part = jnp.take_along_axis(eT[:, gi * 128:(gi + 1) * 128], r, axis=1,
                                   mode="promise_in_bounds")
        mask = (g == gi).astype(jnp.float32)
        acc = part * mask if acc is None else acc + part * mask
    return acc


def _vq_body(x_ref, w_in_ref, b_in_ref, cb_ref, cbm2_ref, cbT_ref, w_out_ref,
             b_out_ref, emb_ref, codes_ref, loss_ref):
    b = pl.program_id(0)
    t = pl.program_id(1)

    @pl.when(jnp.logical_and(b == 0, t == 0))
    def _init():
        loss_ref[...] = jnp.zeros((1, 1), jnp.float32)

    xt = x_ref[0]
    residual = xt
    emb = jnp.zeros_like(xt)
    idx_rows = []
    loss = jnp.float32(0.0)
    for q in range(_NQ):
        if q == 1:
            residual = xt
        v = _mm(w_in_ref[q], residual) + b_in_ref[q]
        e = cb_ref[q]
        ee = jnp.sum(e * e, axis=1, keepdims=True)
        vv = jnp.sum(v * v, axis=0, keepdims=True)
        half = _BINS // 2
        idx_m = []
        for c in range(2):
            vme2 = _mm(cbm2_ref[q, c * half:(c + 1) * half], v)
            d = (vv + vme2) + ee[c * half:(c + 1) * half]
            idx_c = jnp.argmin(d, axis=0, keepdims=True)
            m_c = jnp.min(d, axis=0, keepdims=True)
            idx_m.append((idx_c, m_c))
        (idx_a, m_a), (idx_b, m_b) = idx_m
        take_a = m_a <= m_b
        idx = jnp.where(take_a, idx_a, idx_b + half)
        quant = _gather_rows(cbT_ref[q], idx)
        diff = quant - v
        loss = loss + jnp.sum(diff * diff)
        out = _mm(w_out_ref[q], quant) + b_out_ref[q]
        residual = residual - out
        emb = emb + out
        idx_rows.append(idx.astype(jnp.int32))
    emb_ref[0] = emb
    codes_ref[0] = jnp.concatenate(idx_rows, axis=0)
    loss_ref[...] = loss_ref[...] + loss


def kernel(x, W_in_first, b_in_first, codebook_first, W_out_first, b_out_first,
           W_in_rest, b_in_rest, codebook_rest, W_out_rest, b_out_rest):
    w_in = jnp.concatenate([W_in_first, W_in_rest], axis=0)
    b_in = jnp.concatenate([b_in_first, b_in_rest], axis=0)[:, :, None]
    cb = jnp.concatenate([codebook_first, codebook_rest], axis=0)
    cbm2 = -2.0 * cb
    cbT = jnp.transpose(cb, (0, 2, 1))
    w_out = jnp.concatenate([W_out_first, W_out_rest], axis=0)
    b_out = jnp.concatenate([b_out_first, b_out_rest], axis=0)[:, :, None]

    full = lambda *shape: pl.BlockSpec(shape, lambda b, t: (0,) * len(shape))
    emb, codes, loss = pl.pallas_call(
        _vq_body,
        grid=(_B, _NT),
        in_specs=[
            pl.BlockSpec((1, _C, _TT), lambda b, t: (b, 0, t)),
            full(_NQ, _CDIM, _C),
            full(_NQ, _CDIM, 1),
            full(_NQ, _BINS, _CDIM),
            full(_NQ, _BINS, _CDIM),
            full(_NQ, _CDIM, _BINS),
            full(_NQ, _C, _CDIM),
            full(_NQ, _C, 1),
        ],
        out_specs=[
            pl.BlockSpec((1, _C, _TT), lambda b, t: (b, 0, t)),
            pl.BlockSpec((1, _NQ, _TT), lambda b, t: (b, 0, t)),
            pl.BlockSpec((1, 1), lambda b, t: (0, 0)),
        ],
        out_shape=[
            jax.ShapeDtypeStruct((_B, _C, _T), jnp.float32),
            jax.ShapeDtypeStruct((_B, _NQ, _T), jnp.int32),
            jax.ShapeDtypeStruct((1, 1), jnp.float32),
        ],
    )(x, w_in, b_in, cb, cbm2, cbT, w_out, b_out)

    full_quantized_emb = jnp.transpose(emb, (0, 2, 1))
    full_quantized_codes = jnp.transpose(codes, (0, 2, 1))
    penalty = loss[0, 0] / jnp.float32(_B * _T * _CDIM * _NQ)
    return full_quantized_emb, full_quantized_codes, penalty


# speedup vs baseline: 1.1273x; 1.1273x over previous
import jax
import jax.numpy as jnp
from jax.experimental import pallas as pl

_B, _C, _T = 16, 128, 1500
_BINS = 2048
_CDIM = 32
_NQ = 8
_TT = 1500
_NT = _T // _TT
_NGRP = _BINS // 128

_MM = jax.lax.Precision.DEFAULT


def _mm(a, b):
    return jax.lax.dot_general(a, b, (((1,), (0,)), ((), ())),
                               precision=_MM,
                               preferred_element_type=jnp.float32)


def _gather_rows(eT, idx):
    rows = eT.shape[0]
    r = jnp.broadcast_to(idx % 128, (rows, idx.shape[1]))
    g = idx // 128
    acc = None
    for gi in range(_NGRP):
        part = jnp.take_along_axis(eT[:, gi * 128:(gi + 1) * 128], r, axis=1,
                                   mode="promise_in_bounds")
        mask = (g == gi).astype(jnp.float32)
        acc = part * mask if acc is None else acc + part * mask
    return acc


def _vq_body(x_ref, w_in_ref, b_in_ref, cb_ref, cbm2_ref, cbT_ref, w_out_ref,
             b_out_ref, emb_ref, codes_ref, loss_ref):
    b = pl.program_id(0)
    t = pl.program_id(1)

    @pl.when(jnp.logical_and(b == 0, t == 0))
    def _init():
        loss_ref[...] = jnp.zeros((1, 1), jnp.float32)

    xt = x_ref[0]
    residual = xt
    emb = jnp.zeros_like(xt)
    idx_rows = []
    loss = jnp.float32(0.0)
    for q in range(_NQ):
        if q == 1:
            residual = xt
        v = _mm(w_in_ref[q], residual) + b_in_ref[q]
        e = cb_ref[q]
        ee = jnp.sum(e * e, axis=1, keepdims=True)
        vme2 = _mm(cbm2_ref[q], v)
        vv = jnp.sum(v * v, axis=0, keepdims=True)
        d = (vv + vme2) + ee
        idx = jnp.argmin(d, axis=0, keepdims=True)
        quant = _gather_rows(cbT_ref[q], idx)
        diff = quant - v
        loss = loss + jnp.sum(diff * diff)
        out = _mm(w_out_ref[q], quant) + b_out_ref[q]
        residual = residual - out
        emb = emb + out
        idx_rows.append(idx.astype(jnp.int32))
    emb_ref[0] = emb
    codes_ref[0] = jnp.concatenate(idx_rows, axis=0)
    loss_ref[...] = loss_ref[...] + loss


def kernel(x, W_in_first, b_in_first, codebook_first, W_out_first, b_out_first,
           W_in_rest, b_in_rest, codebook_rest, W_out_rest, b_out_rest):
    w_in = jnp.concatenate([W_in_first, W_in_rest], axis=0)
    b_in = jnp.concatenate([b_in_first, b_in_rest], axis=0)[:, :, None]
    cb = jnp.concatenate([codebook_first, codebook_rest], axis=0)
    cbm2 = -2.0 * cb
    cbT = jnp.transpose(cb, (0, 2, 1))
    w_out = jnp.concatenate([W_out_first, W_out_rest], axis=0)
    b_out = jnp.concatenate([b_out_first, b_out_rest], axis=0)[:, :, None]

    full = lambda *shape: pl.BlockSpec(shape, lambda b, t: (0,) * len(shape))
    emb, codes, loss = pl.pallas_call(
        _vq_body,
        grid=(_B, _NT),
        in_specs=[
            pl.BlockSpec((1, _C, _TT), lambda b, t: (b, 0, t)),
            full(_NQ, _CDIM, _C),
            full(_NQ, _CDIM, 1),
            full(_NQ, _BINS, _CDIM),
            full(_NQ, _BINS, _CDIM),
            full(_NQ, _CDIM, _BINS),
            full(_NQ, _C, _CDIM),
            full(_NQ, _C, 1),
        ],
        out_specs=[
            pl.BlockSpec((1, _C, _TT), lambda b, t: (b, 0, t)),
            pl.BlockSpec((1, _NQ, _TT), lambda b, t: (b, 0, t)),
            pl.BlockSpec((1, 1), lambda b, t: (0, 0)),
        ],
        out_shape=[
            jax.ShapeDtypeStruct((_B, _C, _T), jnp.float32),
            jax.ShapeDtypeStruct((_B, _NQ, _T), jnp.int32),
            jax.ShapeDtypeStruct((1, 1), jnp.float32),
        ],
    )(x, w_in, b_in, cb, cbm2, cbT, w_out, b_out)

    full_quantized_emb = jnp.transpose(emb, (0, 2, 1))
    full_quantized_codes = jnp.transpose(codes, (0, 2, 1))
    penalty = loss[0, 0] / jnp.float32(_B * _T * _CDIM * _NQ)
    return full_quantized_emb, full_quantized_codes, penalty
